# Initial kernel scaffold; baseline (speedup 1.0000x reference)
import jax
import jax.numpy as jnp
from jax.experimental import pallas as pl


def kernel(x, detectors):
    raise NotImplementedError("write your pallas kernel here")


# speedup vs baseline: 3.7511x; 3.7511x over previous
import functools

import jax
import jax.numpy as jnp
from jax import lax
from jax.experimental import pallas as pl
from jax.experimental.pallas import tpu as pltpu
from jax.experimental.pallas import tpu_sc as plsc

B, N = 64, 32768
D, K = 8192, 8
NC, NS = 2, 16
BH = B // NC
DPT = D // NS
CH = 16
ROWS = CH * K
NCHUNK = DPT // CH
RPT = N // NS
RB = 128


def _body(x2, det, out, stat, idxr, idxg, vals, flags, zbuf, sem):
    c = lax.axis_index("c")
    s = lax.axis_index("s")
    cn = c * N

    for i in range(RB):
        zbuf[i, :] = jnp.zeros((BH,), jnp.int16)

    def zero_blk(j, carry):
        pltpu.sync_copy(zbuf, stat.at[pl.ds(s * RPT + j * RB, RB)])
        return carry

    lax.fori_loop(0, RPT // RB, zero_blk, 0)
    plsc.subcore_barrier()

    def chunk(i, carry):
        off = (s * DPT + i * CH) * K
        pltpu.sync_copy(det.at[pl.ds(off, ROWS)], idxr)
        for j in range(ROWS // 16):
            idxg[pl.ds(j * 16, 16)] = idxr[pl.ds(j * 16, 16)] + cn
        pltpu.async_copy(x2.at[idxg], vals, sem).wait()
        for g in range(CH):
            r0 = g * K
            v = [[vals[r0 + k, pl.ds(h * 16, 16)] for h in (0, 1)]
                 for k in range(K)]
            fl = [None, None]
            for h in (0, 1):
                m = v[0][h]
                for k in range(1, K):
                    m = jnp.maximum(m, v[k][h])
                eq0 = v[0][h] == m
                wins = [eq0]
                seen = eq0
                for k in range(1, K):
                    eq = v[k][h] == m
                    wins.append(eq & ~seen)
                    seen = seen | eq
                fl[h] = [jnp.where(w, 0, 1).astype(jnp.int32) for w in wins]
            for k in range(K):
                packed = fl[0][k] | lax.shift_left(fl[1][k], 16)
                flags[r0 + k, :] = plsc.bitcast(packed, jnp.int16)
        pltpu.sync_copy(flags, stat.at[idxr], add=True)
        return carry

    lax.fori_loop(0, NCHUNK, chunk, 0)
    plsc.subcore_barrier()

    def out_blk(i, carry):
        r0 = s * RPT + i * RB
        pltpu.sync_copy(stat.at[pl.ds(r0, RB)], flags)
        pltpu.sync_copy(x2.at[pl.ds(cn + r0, RB)], vals)
        for r in range(RB):
            w = plsc.bitcast(flags[r, :], jnp.int32)
            a = w & 0xFFFF
            b = lax.shift_right_logical(w, 16)
            x0 = vals[r, pl.ds(0, 16)]
            x1 = vals[r, pl.ds(16, 16)]
            vals[r, pl.ds(0, 16)] = jnp.where(a == 0, x0, 0.0)
            vals[r, pl.ds(16, 16)] = jnp.where(b == 0, x1, 0.0)
        pltpu.sync_copy(vals, out.at[pl.ds(cn + r0, RB)])
        return carry

    lax.fori_loop(0, RPT // RB, out_blk, 0)


_sc_call = functools.partial(
    pl.kernel,
    out_type=jax.ShapeDtypeStruct((2 * N, BH), jnp.float32),
    mesh=plsc.VectorSubcoreMesh(core_axis_name="c", subcore_axis_name="s"),
    compiler_params=pltpu.CompilerParams(
        needs_layout_passes=False, use_tc_tiling_on_sc=False),
    scratch_types=[
        pltpu.VMEM_SHARED((N, BH), jnp.int16),
        pltpu.VMEM((ROWS,), jnp.int32),
        pltpu.VMEM((ROWS,), jnp.int32),
        pltpu.VMEM((ROWS, BH), jnp.float32),
        pltpu.VMEM((ROWS, BH), jnp.int16),
        pltpu.VMEM((RB, BH), jnp.int16),
        pltpu.SemaphoreType.DMA,
    ],
)(_body)


@jax.jit
def kernel(x, detectors):
    x2 = x.reshape(NC, BH, N).transpose(0, 2, 1).reshape(NC * N, BH)
    det = detectors.reshape(-1)
    out2 = _sc_call(x2, det)
    return out2.reshape(NC, N, BH).transpose(0, 2, 1).reshape(B, N)
